# Initial kernel scaffold; baseline (speedup 1.0000x reference)
import jax
import jax.numpy as jnp
from jax.experimental import pallas as pl


def kernel(E, P, W_enc, b_enc, W_M1, b_M1, W_M2, b_M2, W_U, b_U, W_dec, b_dec, W_mask, b_mask, W_q, b_q, W_k, b_k):
    raise NotImplementedError("write your pallas kernel here")


# speedup vs baseline: 1.1644x; 1.1644x over previous
import jax
import jax.numpy as jnp
from jax.experimental import pallas as pl

L, B, K = 4, 8, 128
ENC, HID, Q = 128, 128, 64
NEG = float("-inf")
BIGNEG = -3.0e38

def _bdot(x, w):
    return jnp.dot(x.astype(jnp.bfloat16), w.astype(jnp.bfloat16),
                   preferred_element_type=jnp.float32)

def _bf(x):
    return x.astype(jnp.bfloat16).astype(jnp.float32)


def _pgn_body(E_ref, Pf_ref, WencE_ref, WencH_ref, benc_ref,
              WM1_ref, bM1_ref, WM2_ref, bM2_ref,
              WUz_ref, WUm_ref, bU_ref,
              wdz_ref, wdh_ref, bdec_ref,
              wmz_ref, wmh_ref, bmask_ref,
              Wq_ref, bq_ref, Wk_ref, bk_ref,
              Y_ref, A_ref, M_ref):
    f32 = jnp.float32
    we = WencE_ref[...]
    WencH = WencH_ref[...]
    benc = benc_ref[...]
    WM1 = WM1_ref[...]
    bM1 = bM1_ref[...]
    WM2 = WM2_ref[...]
    bM2 = bM2_ref[...]
    WUz = WUz_ref[...]
    WUm = WUm_ref[...]
    bU = bU_ref[...]
    wdz = wdz_ref[...]
    wdh = wdh_ref[...]
    bdec = bdec_ref[...]
    wmz = wmz_ref[...]
    wmh = wmh_ref[...]
    bmask = bmask_ref[...]
    Wq = Wq_ref[...]
    bq = bq_ref[...]
    Wk = Wk_ref[...]
    bk = bk_ref[...]

    h2 = jnp.zeros((B * K, HID), f32)
    for t in range(L):
        E2 = E_ref[t].reshape(B * K, 2)
        E2b = _bf(E2)
        web = _bf(we)
        z2 = (E2b[:, 0:1] * web[0:1, :] + E2b[:, 1:2] * web[1:2, :]
              + jnp.dot(h2, WencH, preferred_element_type=f32, precision=jax.lax.Precision.HIGH) + benc)
        m1z = jnp.dot(z2, WM1, preferred_element_type=f32, precision=jax.lax.Precision.HIGH) + bM1
        m2z = jnp.dot(z2, WM2, preferred_element_type=f32, precision=jax.lax.Precision.HIGH) + bM2
        m2z3 = m2z.reshape(B, K, ENC)

        mx_list = []
        for b in range(B):
            contrib = Pf_ref[t, b][:, :, None] + m2z3[b][:, None, :]
            mx_list.append(jnp.max(contrib, axis=0))
        mx2 = jnp.concatenate(mx_list, axis=0)
        m2 = jnp.where(mx2 < -1e37, NEG, jax.nn.relu(m1z + mx2))

        h2 = jax.nn.relu(jnp.dot(z2, WUz, preferred_element_type=f32, precision=jax.lax.Precision.HIGH)
                         + jnp.dot(m2, WUm, preferred_element_type=f32, precision=jax.lax.Precision.HIGH) + bU)

        z3 = z2.reshape(B, K, ENC)
        h3 = h2.reshape(B, K, HID)
        maxz = jnp.max(z3, axis=1)
        maxh = jnp.max(h3, axis=1)
        ypre = (jnp.sum(_bf(maxz) * _bf(wdz), axis=-1, keepdims=True)
                + jnp.sum(_bf(maxh) * _bf(wdh), axis=-1, keepdims=True) + bdec)
        Y_ref[t] = jax.nn.sigmoid(ypre)

        q2 = jnp.dot(h2, Wq, preferred_element_type=f32, precision=jax.lax.Precision.HIGH) + bq
        k2 = jnp.dot(h2, Wk, preferred_element_type=f32, precision=jax.lax.Precision.HIGH) + bk
        q3 = q2.reshape(B, K, Q)
        k3 = k2.reshape(B, K, Q)
        for b in range(B):
            A_ref[t, b] = jax.lax.dot_general(
                q3[b], k3[b], (((1,), (1,)), ((), ())),
                preferred_element_type=f32, precision=jax.lax.Precision.HIGH)

        mm_pre = (jnp.sum(_bf(z3) * _bf(wmz)[None], axis=-1)
                  + jnp.sum(_bf(h3) * _bf(wmh)[None], axis=-1) + bmask)
        M_ref[t] = jax.nn.sigmoid(mm_pre)


def _pgn_call(E, Pf, *weights):
    out_shape = [
        jax.ShapeDtypeStruct((L, B, 1), jnp.float32),
        jax.ShapeDtypeStruct((L, B, K, K), jnp.float32),
        jax.ShapeDtypeStruct((L, B, K), jnp.float32),
    ]
    return pl.pallas_call(_pgn_body, out_shape=out_shape)(E, Pf, *weights)


def kernel(E, P, W_enc, b_enc, W_M1, b_M1, W_M2, b_M2, W_U, b_U,
           W_dec, b_dec, W_mask, b_mask, W_q, b_q, W_k, b_k):
    f32 = jnp.float32
    Pf = jnp.where(P, f32(0.0), f32(BIGNEG))
    weights = (
        W_enc[:2], W_enc[2:], b_enc.reshape(1, ENC),
        W_M1, b_M1.reshape(1, ENC), W_M2, b_M2.reshape(1, ENC),
        W_U[:ENC], W_U[ENC:], b_U.reshape(1, HID),
        W_dec[:ENC].reshape(1, ENC), W_dec[ENC:].reshape(1, HID),
        b_dec.reshape(1, 1),
        W_mask[:ENC].reshape(1, ENC), W_mask[ENC:].reshape(1, HID),
        b_mask.reshape(1, 1),
        W_q, b_q.reshape(1, Q), W_k, b_k.reshape(1, Q),
    )
    Y, A, M = _pgn_call(E, Pf, *weights)
    return Y[..., 0], A, M
